# speedup vs baseline: 5.0786x; 5.0786x over previous
import functools
import math

import jax
import jax.numpy as jnp
from jax import lax
from jax.experimental import pallas as pl
from jax.experimental.pallas import tpu as pltpu
from jax.experimental.pallas import tpu_sc as plsc

N_TOKEN = 100000
D_MODEL = 128
SCALE = math.sqrt(float(D_MODEL))

_info = plsc.get_sparse_core_info()
NC = _info.num_cores
NS = _info.num_subcores
NW = NC * NS

B_TOTAL = 4096 * 200
B_PER_W = B_TOTAL // NW
CH = 128
N_CH = B_PER_W // CH


@functools.partial(
    pl.kernel,
    out_type=jax.ShapeDtypeStruct((B_TOTAL, D_MODEL), jnp.float32),
    mesh=plsc.VectorSubcoreMesh(core_axis_name="c", subcore_axis_name="s"),
    scratch_types=[
        pltpu.VMEM((N_CH, CH), jnp.int32),
        pltpu.VMEM((CH, D_MODEL), jnp.float32),
        pltpu.SemaphoreType.DMA,
    ],
)
def _emb_kernel(x_hbm, table_hbm, out_hbm, idx_v, rows_v, sem):
    wid = lax.axis_index("s") * NC + lax.axis_index("c")
    base = wid * B_PER_W

    pltpu.sync_copy(x_hbm.at[wid], idx_v)

    def chunk_body(j, carry):
        pltpu.async_copy(table_hbm.at[idx_v.at[j]], rows_v, sem).wait()

        def row_body(r, c):
            for p in range(D_MODEL // 16):
                sl = pl.ds(p * 16, 16)
                rows_v[r, sl] = rows_v[r, sl] * SCALE
            return c

        lax.fori_loop(0, CH, row_body, 0)

        pltpu.sync_copy(rows_v, out_hbm.at[pl.ds(base + j * CH, CH)])
        return carry

    lax.fori_loop(0, N_CH, chunk_body, 0)


def kernel(x, emb_weight):
    x_flat = x.reshape(NW, N_CH, CH).astype(jnp.int32)
    out = _emb_kernel(x_flat, emb_weight)
    return out.reshape(x.shape[0], x.shape[1], D_MODEL)


# speedup vs baseline: 9.2506x; 1.8215x over previous
import functools
import math

import jax
import jax.numpy as jnp
from jax import lax
from jax.experimental import pallas as pl
from jax.experimental.pallas import tpu as pltpu
from jax.experimental.pallas import tpu_sc as plsc

N_TOKEN = 100000
D_MODEL = 128
SCALE = math.sqrt(float(D_MODEL))

_info = plsc.get_sparse_core_info()
NC = _info.num_cores
NS = _info.num_subcores
NW = NC * NS

B_TOTAL = 4096 * 200
B_PER_W = B_TOTAL // NW
CH = 128
N_CH = B_PER_W // CH


@functools.partial(
    pl.kernel,
    out_type=jax.ShapeDtypeStruct((B_TOTAL, D_MODEL), jnp.float32),
    mesh=plsc.VectorSubcoreMesh(core_axis_name="c", subcore_axis_name="s"),
    scratch_types=[
        pltpu.VMEM((N_CH, CH), jnp.int32),
        pltpu.VMEM((CH, D_MODEL), jnp.float32),
        pltpu.VMEM((CH, D_MODEL), jnp.float32),
        pltpu.VMEM((CH, D_MODEL), jnp.float32),
        pltpu.VMEM((CH, D_MODEL), jnp.float32),
        pltpu.SemaphoreType.DMA,
        pltpu.SemaphoreType.DMA,
        pltpu.SemaphoreType.DMA,
        pltpu.SemaphoreType.DMA,
    ],
)
def _emb_kernel(x_hbm, table_hbm, out_hbm, idx_v, g0, g1, s0, s1,
                gsem0, gsem1, ssem0, ssem1):
    wid = lax.axis_index("s") * NC + lax.axis_index("c")
    base = wid * B_PER_W
    gbufs, sbufs = (g0, g1), (s0, s1)
    gsems, ssems = (gsem0, gsem1), (ssem0, ssem1)

    pltpu.sync_copy(x_hbm.at[wid], idx_v)

    for b in range(2):
        pltpu.make_async_copy(
            table_hbm.at[idx_v.at[b]], gbufs[b], gsems[b]).start()

    def outer(gi, carry):
        for b in range(2):
            j = gi * 2 + b
            gb, sb = gbufs[b], sbufs[b]

            pltpu.make_async_copy(
                table_hbm.at[idx_v.at[j]], gb, gsems[b]).wait()

            @pl.when(j >= 2)
            def _():
                pltpu.make_async_copy(
                    sb, out_hbm.at[pl.ds(base + (j - 2) * CH, CH)],
                    ssems[b]).wait()

            @plsc.parallel_loop(0, CH, 1, unroll=4)
            def _(r):
                for p in range(D_MODEL // 16):
                    sl = pl.ds(p * 16, 16)
                    sb[r, sl] = gb[r, sl] * SCALE

            @pl.when(j + 2 < N_CH)
            def _():
                pltpu.make_async_copy(
                    table_hbm.at[idx_v.at[j + 2]], gb, gsems[b]).start()

            pltpu.make_async_copy(
                sb, out_hbm.at[pl.ds(base + j * CH, CH)], ssems[b]).start()
        return carry

    lax.fori_loop(0, N_CH // 2, outer, 0)

    for b in range(2):
        j = N_CH - 2 + b
        pltpu.make_async_copy(
            sbufs[b], out_hbm.at[pl.ds(base + j * CH, CH)], ssems[b]).wait()


def kernel(x, emb_weight):
    x_flat = x.reshape(NW, N_CH, CH).astype(jnp.int32)
    out = _emb_kernel(x_flat, emb_weight)
    return out.reshape(x.shape[0], x.shape[1], D_MODEL)
